# baseline (speedup 1.0000x reference)
import functools

import jax
import jax.numpy as jnp
from jax import lax
from jax.experimental import pallas as pl
from jax.experimental.pallas import tpu as pltpu
from jax.experimental.pallas import tpu_sc as plsc

N_NODES = 10000
N_EDGES = 320000
DIM = 128

_N_TILES = 32
_CHUNK = 128
_N_CHUNKS = N_EDGES // _CHUNK
_NBUF = 3
_ITERS = 81
_WSLAB = 80
_N_SLABS = N_NODES // _WSLAB
_SLAB_ITERS = -(-_N_SLABS // 16)

_sc_mesh = plsc.VectorSubcoreMesh(core_axis_name="c", subcore_axis_name="s")


@functools.partial(
    pl.kernel,
    mesh=_sc_mesh,
    out_type=jax.ShapeDtypeStruct((2, N_NODES, DIM), jnp.float32),
    scratch_types=(
        [pltpu.VMEM((_CHUNK,), jnp.int32)] * _NBUF
        + [pltpu.VMEM((_CHUNK,), jnp.int32)] * _NBUF
        + [pltpu.VMEM((_CHUNK, DIM), jnp.float32)] * _NBUF
        + [pltpu.VMEM_SHARED((N_NODES, DIM), jnp.float32)]
        + [pltpu.SemaphoreType.DMA] * (3 * _NBUF + 1)
    ),
)
def _sc_agg(table_hbm, ei_hbm, zeros_hbm, out_hbm, *scr):
    src_b = scr[0:_NBUF]
    dst_b = scr[_NBUF:2 * _NBUF]
    rows_b = scr[2 * _NBUF:3 * _NBUF]
    acc_sh = scr[3 * _NBUF]
    isem = scr[3 * _NBUF + 1:3 * _NBUF + 1 + _NBUF]
    gsem = scr[3 * _NBUF + 1 + _NBUF:3 * _NBUF + 1 + 2 * _NBUF]
    ssem = scr[3 * _NBUF + 1 + 2 * _NBUF:3 * _NBUF + 1 + 3 * _NBUF]
    zsem = scr[3 * _NBUF + 1 + 3 * _NBUF]

    cid = lax.axis_index("c")
    sid = lax.axis_index("s")
    wid = cid * 16 + sid

    def _ci(k):
        return k * _N_TILES + wid

    def _valid(k):
        return _ci(k) < _N_CHUNKS

    def _src_slice(k):
        return ei_hbm.at[0, pl.ds(_ci(k) * _CHUNK, _CHUNK)]

    def _dst_slice(k):
        return ei_hbm.at[1, pl.ds(_ci(k) * _CHUNK, _CHUNK)]

    def _slab_ok(j):
        return (j * 16 + sid) < _N_SLABS

    def _slab_r0(j):
        return pl.multiple_of((j * 16 + sid) * _WSLAB, 8)

    for k0 in range(_NBUF):
        pltpu.async_copy(_src_slice(k0), src_b[k0], isem[k0])

    def _zfill(j):
        return pltpu.make_async_copy(zeros_hbm,
                                     acc_sh.at[pl.ds(_slab_r0(j), _WSLAB)],
                                     zsem)

    for j in range(_SLAB_ITERS):
        @pl.when(_slab_ok(j))
        def _():
            _zfill(j).start()

    def _edge_round(p, carry):
        for h in range(_NBUF):
            k = p * _NBUF + h
            b = h
            bp = (h + _NBUF - 1) % _NBUF

            @pl.when((k >= _NBUF) & _valid(k - _NBUF))
            def _():
                pltpu.make_async_copy(
                    table_hbm.at[pl.ds(0, _CHUNK)], rows_b[b], ssem[b]
                ).wait()

            @pl.when(_valid(k))
            def _():
                pltpu.make_async_copy(_src_slice(k), src_b[b], isem[b]).wait()
                pltpu.async_copy(table_hbm.at[src_b[b]], rows_b[b], gsem[b])
                pltpu.async_copy(_dst_slice(k), dst_b[b], ssem[b])

            @pl.when((k >= 1) & _valid(k - 1))
            def _():
                pltpu.make_async_copy(table_hbm.at[src_b[bp]], rows_b[bp],
                                      gsem[bp]).wait()

                @pl.when(k == 1)
                def _():
                    for j in range(_SLAB_ITERS):
                        @pl.when(_slab_ok(j))
                        def _():
                            _zfill(j).wait()
                    plsc.subcore_barrier()

                @pl.when(_valid(k + _NBUF - 1))
                def _():
                    pltpu.async_copy(_src_slice(k + _NBUF - 1), src_b[bp],
                                     isem[bp])

                pltpu.make_async_copy(_dst_slice(k - 1), dst_b[bp],
                                      ssem[bp]).wait()
                pltpu.async_copy(rows_b[bp], acc_sh.at[dst_b[bp]], ssem[bp],
                                 add=True)

        return carry

    lax.fori_loop(0, (_ITERS + 2 * _NBUF) // _NBUF, _edge_round, 0)

    plsc.subcore_barrier()

    def _stage(j):
        return rows_b[j % _NBUF].at[pl.ds(0, _WSLAB)]

    def _fetch(j):
        return pltpu.make_async_copy(acc_sh.at[pl.ds(_slab_r0(j), _WSLAB)],
                                     _stage(j), gsem[j % _NBUF])

    def _store(j):
        return pltpu.make_async_copy(_stage(j),
                                     out_hbm.at[cid, pl.ds(_slab_r0(j), _WSLAB)],
                                     ssem[j % _NBUF])

    for j in range(_SLAB_ITERS + 1):
        if j < _SLAB_ITERS:
            if j >= _NBUF:
                @pl.when(_slab_ok(j - _NBUF))
                def _():
                    _store(j - _NBUF).wait()

            @pl.when(_slab_ok(j))
            def _():
                _fetch(j).start()

        if j >= 1:
            @pl.when(_slab_ok(j - 1))
            def _():
                _fetch(j - 1).wait()
                _store(j - 1).start()

    for j in range(_SLAB_ITERS - _NBUF, _SLAB_ITERS):
        @pl.when(_slab_ok(j))
        def _():
            _store(j).wait()


_BLK = 1000


def _mlp1_body(x_ref, a0_ref, a1_ref, w1_ref, b1_ref, w2_ref, b2_ref, o_ref):
    s = x_ref[...] + a0_ref[0] + a1_ref[0]
    h = jnp.dot(s, w1_ref[...], preferred_element_type=jnp.float32) + b1_ref[...]
    h = jnp.maximum(h, 0.0)
    o_ref[...] = jnp.dot(h, w2_ref[...], preferred_element_type=jnp.float32) + b2_ref[...]


def _mlp1(x, acc, W1, b1, W2, b2):
    grid = (N_NODES // _BLK,)
    return pl.pallas_call(
        _mlp1_body,
        grid=grid,
        in_specs=[
            pl.BlockSpec((_BLK, DIM), lambda i: (i, 0)),
            pl.BlockSpec((1, _BLK, DIM), lambda i: (0, i, 0)),
            pl.BlockSpec((1, _BLK, DIM), lambda i: (1, i, 0)),
            pl.BlockSpec((DIM, DIM), lambda i: (0, 0)),
            pl.BlockSpec((1, DIM), lambda i: (0, 0)),
            pl.BlockSpec((DIM, DIM), lambda i: (0, 0)),
            pl.BlockSpec((1, DIM), lambda i: (0, 0)),
        ],
        out_specs=pl.BlockSpec((_BLK, DIM), lambda i: (i, 0)),
        out_shape=jax.ShapeDtypeStruct((N_NODES, DIM), jnp.float32),
    )(x, acc, acc, W1, b1.reshape(1, DIM), W2, b2.reshape(1, DIM))


def _mlp2_body(h_ref, a0_ref, a1_ref, w3_ref, b3_ref, w4_ref, b4_ref,
               wf_ref, bf_ref, o_ref, acc_ref):
    s = h_ref[...] + a0_ref[0] + a1_ref[0]
    z = jnp.dot(s, w3_ref[...], preferred_element_type=jnp.float32) + b3_ref[...]
    part = jnp.sum(jnp.maximum(z, 0.0), axis=0, keepdims=True)

    @pl.when(pl.program_id(0) == 0)
    def _():
        acc_ref[...] = jnp.zeros_like(acc_ref)

    acc_ref[...] += part

    @pl.when(pl.program_id(0) == pl.num_programs(0) - 1)
    def _():
        pooled = (
            jnp.dot(acc_ref[...] * (1.0 / N_NODES), w4_ref[...],
                    preferred_element_type=jnp.float32)
            + b4_ref[...]
        )
        o_ref[...] = (
            jnp.dot(pooled, wf_ref[...], preferred_element_type=jnp.float32)
            + bf_ref[...]
        )


def _mlp2(h, acc, W3, b3, W4, b4, Wf, bf):
    grid = (N_NODES // _BLK,)
    return pl.pallas_call(
        _mlp2_body,
        grid=grid,
        in_specs=[
            pl.BlockSpec((_BLK, DIM), lambda i: (i, 0)),
            pl.BlockSpec((1, _BLK, DIM), lambda i: (0, i, 0)),
            pl.BlockSpec((1, _BLK, DIM), lambda i: (1, i, 0)),
            pl.BlockSpec((DIM, DIM), lambda i: (0, 0)),
            pl.BlockSpec((1, DIM), lambda i: (0, 0)),
            pl.BlockSpec((DIM, DIM), lambda i: (0, 0)),
            pl.BlockSpec((1, DIM), lambda i: (0, 0)),
            pl.BlockSpec((DIM, DIM), lambda i: (0, 0)),
            pl.BlockSpec((1, DIM), lambda i: (0, 0)),
        ],
        out_specs=pl.BlockSpec((1, DIM), lambda i: (0, 0)),
        out_shape=jax.ShapeDtypeStruct((1, DIM), jnp.float32),
        scratch_shapes=[pltpu.VMEM((1, DIM), jnp.float32)],
    )(h, acc, acc, W3, b3.reshape(1, DIM), W4, b4.reshape(1, DIM),
      Wf, bf.reshape(1, DIM))


def kernel(x, edge_index, W1, b1, W2, b2, W3, b3, W4, b4, Wf, bf):
    ei = edge_index.astype(jnp.int32)
    zrows = jnp.zeros((_WSLAB, DIM), jnp.float32)
    acc1 = _sc_agg(x, ei, zrows)
    h1 = _mlp1(x, acc1, W1, b1, W2, b2)
    acc2 = _sc_agg(h1, ei, zrows)
    out = _mlp2(h1, acc2, W3, b3, W4, b4, Wf, bf)
    return out.reshape(DIM)


# speedup vs baseline: 1.1623x; 1.1623x over previous
import functools

import jax
import jax.numpy as jnp
from jax import lax
from jax.experimental import pallas as pl
from jax.experimental.pallas import tpu as pltpu
from jax.experimental.pallas import tpu_sc as plsc

N_NODES = 10000
N_EDGES = 320000
DIM = 128

_N_TILES = 32
_CHUNK = 128
_N_CHUNKS = N_EDGES // _CHUNK
_NBUF = 3
_ITERS = 81
_WSLAB = 80
_N_SLABS = N_NODES // _WSLAB
_SLAB_ITERS = -(-_N_SLABS // 16)

_sc_mesh = plsc.VectorSubcoreMesh(core_axis_name="c", subcore_axis_name="s")


@functools.partial(
    pl.kernel,
    mesh=_sc_mesh,
    out_type=jax.ShapeDtypeStruct((2, N_NODES, DIM), jnp.float32),
    scratch_types=(
        [pltpu.VMEM((_CHUNK,), jnp.int32)] * _NBUF
        + [pltpu.VMEM((_CHUNK,), jnp.int32)] * _NBUF
        + [pltpu.VMEM((_CHUNK, DIM), jnp.float32)] * _NBUF
        + [pltpu.VMEM_SHARED((N_NODES, DIM), jnp.float32)]
        + [pltpu.SemaphoreType.DMA] * (3 * _NBUF)
    ),
)
def _sc_agg(table_hbm, ei_hbm, out_hbm, *scr):
    src_b = scr[0:_NBUF]
    dst_b = scr[_NBUF:2 * _NBUF]
    rows_b = scr[2 * _NBUF:3 * _NBUF]
    acc_sh = scr[3 * _NBUF]
    isem = scr[3 * _NBUF + 1:3 * _NBUF + 1 + _NBUF]
    gsem = scr[3 * _NBUF + 1 + _NBUF:3 * _NBUF + 1 + 2 * _NBUF]
    ssem = scr[3 * _NBUF + 1 + 2 * _NBUF:3 * _NBUF + 1 + 3 * _NBUF]

    cid = lax.axis_index("c")
    sid = lax.axis_index("s")
    wid = cid * 16 + sid

    def _ci(k):
        return k * _N_TILES + wid

    def _valid(k):
        return _ci(k) < _N_CHUNKS

    def _src_slice(k):
        return ei_hbm.at[0, pl.ds(_ci(k) * _CHUNK, _CHUNK)]

    def _dst_slice(k):
        return ei_hbm.at[1, pl.ds(_ci(k) * _CHUNK, _CHUNK)]

    def _slab_ok(j):
        return (j * 16 + sid) < _N_SLABS

    def _slab_r0(j):
        return pl.multiple_of((j * 16 + sid) * _WSLAB, 8)

    for k0 in range(_NBUF):
        pltpu.async_copy(_src_slice(k0), src_b[k0], isem[k0])

    def _zero_row(r, carry):
        for j in range(DIM // 16):
            rows_b[0][r, pl.ds(j * 16, 16)] = jnp.zeros((16,), jnp.float32)
        return carry

    lax.fori_loop(0, _WSLAB, _zero_row, 0)
    for j in range(_SLAB_ITERS):
        @pl.when(_slab_ok(j))
        def _():
            pltpu.async_copy(rows_b[0].at[pl.ds(0, _WSLAB)],
                             acc_sh.at[pl.ds(_slab_r0(j), _WSLAB)], gsem[0])
    for j in range(_SLAB_ITERS):
        @pl.when(_slab_ok(j))
        def _():
            pltpu.make_async_copy(rows_b[0].at[pl.ds(0, _WSLAB)],
                                  acc_sh.at[pl.ds(_slab_r0(j), _WSLAB)],
                                  gsem[0]).wait()
    plsc.subcore_barrier()

    def _edge_round(p, carry):
        for h in range(_NBUF):
            k = p * _NBUF + h
            b = h
            bp = (h + _NBUF - 1) % _NBUF

            @pl.when((k >= _NBUF) & _valid(k - _NBUF))
            def _():
                pltpu.make_async_copy(
                    table_hbm.at[pl.ds(0, _CHUNK)], rows_b[b], ssem[b]
                ).wait()

            @pl.when(_valid(k))
            def _():
                pltpu.make_async_copy(_src_slice(k), src_b[b], isem[b]).wait()
                pltpu.async_copy(table_hbm.at[src_b[b]], rows_b[b], gsem[b])
                pltpu.async_copy(_dst_slice(k), dst_b[b], ssem[b])

            @pl.when((k >= 1) & _valid(k - 1))
            def _():
                pltpu.make_async_copy(table_hbm.at[src_b[bp]], rows_b[bp],
                                      gsem[bp]).wait()

                @pl.when(_valid(k + _NBUF - 1))
                def _():
                    pltpu.async_copy(_src_slice(k + _NBUF - 1), src_b[bp],
                                     isem[bp])

                pltpu.make_async_copy(_dst_slice(k - 1), dst_b[bp],
                                      ssem[bp]).wait()
                pltpu.async_copy(rows_b[bp], acc_sh.at[dst_b[bp]], ssem[bp],
                                 add=True)

        return carry

    lax.fori_loop(0, (_ITERS + 2 * _NBUF) // _NBUF, _edge_round, 0)

    plsc.subcore_barrier()

    def _stage(j):
        return rows_b[j % _NBUF].at[pl.ds(0, _WSLAB)]

    def _fetch(j):
        return pltpu.make_async_copy(acc_sh.at[pl.ds(_slab_r0(j), _WSLAB)],
                                     _stage(j), gsem[j % _NBUF])

    def _store(j):
        return pltpu.make_async_copy(_stage(j),
                                     out_hbm.at[cid, pl.ds(_slab_r0(j), _WSLAB)],
                                     ssem[j % _NBUF])

    for j in range(_SLAB_ITERS + 1):
        if j < _SLAB_ITERS:
            if j >= _NBUF:
                @pl.when(_slab_ok(j - _NBUF))
                def _():
                    _store(j - _NBUF).wait()

            @pl.when(_slab_ok(j))
            def _():
                _fetch(j).start()

        if j >= 1:
            @pl.when(_slab_ok(j - 1))
            def _():
                _fetch(j - 1).wait()
                _store(j - 1).start()

    for j in range(_SLAB_ITERS - _NBUF, _SLAB_ITERS):
        @pl.when(_slab_ok(j))
        def _():
            _store(j).wait()


_BLK = 2000


def _mlp1_body(x_ref, a0_ref, a1_ref, w1_ref, b1_ref, w2_ref, b2_ref, o_ref):
    s = x_ref[...] + a0_ref[0] + a1_ref[0]
    h = jnp.dot(s, w1_ref[...], preferred_element_type=jnp.float32) + b1_ref[...]
    h = jnp.maximum(h, 0.0)
    o_ref[...] = jnp.dot(h, w2_ref[...], preferred_element_type=jnp.float32) + b2_ref[...]


def _mlp1(x, acc, W1, b1, W2, b2):
    grid = (N_NODES // _BLK,)
    return pl.pallas_call(
        _mlp1_body,
        grid=grid,
        in_specs=[
            pl.BlockSpec((_BLK, DIM), lambda i: (i, 0)),
            pl.BlockSpec((1, _BLK, DIM), lambda i: (0, i, 0)),
            pl.BlockSpec((1, _BLK, DIM), lambda i: (1, i, 0)),
            pl.BlockSpec((DIM, DIM), lambda i: (0, 0)),
            pl.BlockSpec((1, DIM), lambda i: (0, 0)),
            pl.BlockSpec((DIM, DIM), lambda i: (0, 0)),
            pl.BlockSpec((1, DIM), lambda i: (0, 0)),
        ],
        out_specs=pl.BlockSpec((_BLK, DIM), lambda i: (i, 0)),
        out_shape=jax.ShapeDtypeStruct((N_NODES, DIM), jnp.float32),
    )(x, acc, acc, W1, b1.reshape(1, DIM), W2, b2.reshape(1, DIM))


def _mlp2_body(h_ref, a0_ref, a1_ref, w3_ref, b3_ref, w4_ref, b4_ref,
               wf_ref, bf_ref, o_ref, acc_ref):
    s = h_ref[...] + a0_ref[0] + a1_ref[0]
    z = jnp.dot(s, w3_ref[...], preferred_element_type=jnp.float32) + b3_ref[...]
    part = jnp.sum(jnp.maximum(z, 0.0), axis=0, keepdims=True)

    @pl.when(pl.program_id(0) == 0)
    def _():
        acc_ref[...] = jnp.zeros_like(acc_ref)

    acc_ref[...] += part

    @pl.when(pl.program_id(0) == pl.num_programs(0) - 1)
    def _():
        pooled = (
            jnp.dot(acc_ref[...] * (1.0 / N_NODES), w4_ref[...],
                    preferred_element_type=jnp.float32)
            + b4_ref[...]
        )
        o_ref[...] = (
            jnp.dot(pooled, wf_ref[...], preferred_element_type=jnp.float32)
            + bf_ref[...]
        )


def _mlp2(h, acc, W3, b3, W4, b4, Wf, bf):
    grid = (N_NODES // _BLK,)
    return pl.pallas_call(
        _mlp2_body,
        grid=grid,
        in_specs=[
            pl.BlockSpec((_BLK, DIM), lambda i: (i, 0)),
            pl.BlockSpec((1, _BLK, DIM), lambda i: (0, i, 0)),
            pl.BlockSpec((1, _BLK, DIM), lambda i: (1, i, 0)),
            pl.BlockSpec((DIM, DIM), lambda i: (0, 0)),
            pl.BlockSpec((1, DIM), lambda i: (0, 0)),
            pl.BlockSpec((DIM, DIM), lambda i: (0, 0)),
            pl.BlockSpec((1, DIM), lambda i: (0, 0)),
            pl.BlockSpec((DIM, DIM), lambda i: (0, 0)),
            pl.BlockSpec((1, DIM), lambda i: (0, 0)),
        ],
        out_specs=pl.BlockSpec((1, DIM), lambda i: (0, 0)),
        out_shape=jax.ShapeDtypeStruct((1, DIM), jnp.float32),
        scratch_shapes=[pltpu.VMEM((1, DIM), jnp.float32)],
    )(h, acc, acc, W3, b3.reshape(1, DIM), W4, b4.reshape(1, DIM),
      Wf, bf.reshape(1, DIM))


def kernel(x, edge_index, W1, b1, W2, b2, W3, b3, W4, b4, Wf, bf):
    ei = edge_index.astype(jnp.int32)
    acc1 = _sc_agg(x, ei)
    h1 = _mlp1(x, acc1, W1, b1, W2, b2)
    acc2 = _sc_agg(h1, ei)
    out = _mlp2(h1, acc2, W3, b3, W4, b4, Wf, bf)
    return out.reshape(DIM)


# speedup vs baseline: 1.1712x; 1.0076x over previous
import functools

import jax
import jax.numpy as jnp
from jax import lax
from jax.experimental import pallas as pl
from jax.experimental.pallas import tpu as pltpu
from jax.experimental.pallas import tpu_sc as plsc

N_NODES = 10000
N_EDGES = 320000
DIM = 128

_N_TILES = 32
_CHUNK = 128
_N_CHUNKS = N_EDGES // _CHUNK
_NBUF = 3
_ITERS = 81
_WSLAB = 80
_N_SLABS = N_NODES // _WSLAB
_SLAB_ITERS = -(-_N_SLABS // 16)

_sc_mesh = plsc.VectorSubcoreMesh(core_axis_name="c", subcore_axis_name="s")


@functools.partial(
    pl.kernel,
    mesh=_sc_mesh,
    out_type=jax.ShapeDtypeStruct((2, N_NODES, DIM), jnp.float32),
    scratch_types=(
        [pltpu.VMEM((_CHUNK,), jnp.int32)] * _NBUF
        + [pltpu.VMEM((_CHUNK,), jnp.int32)] * _NBUF
        + [pltpu.VMEM((_CHUNK, DIM), jnp.float32)] * _NBUF
        + [pltpu.VMEM_SHARED((N_NODES, DIM), jnp.float32)]
        + [pltpu.SemaphoreType.DMA] * (3 * _NBUF)
    ),
)
def _sc_agg(table_hbm, ei_hbm, out_hbm, *scr):
    src_b = scr[0:_NBUF]
    dst_b = scr[_NBUF:2 * _NBUF]
    rows_b = scr[2 * _NBUF:3 * _NBUF]
    acc_sh = scr[3 * _NBUF]
    isem = scr[3 * _NBUF + 1:3 * _NBUF + 1 + _NBUF]
    gsem = scr[3 * _NBUF + 1 + _NBUF:3 * _NBUF + 1 + 2 * _NBUF]
    ssem = scr[3 * _NBUF + 1 + 2 * _NBUF:3 * _NBUF + 1 + 3 * _NBUF]

    cid = lax.axis_index("c")
    sid = lax.axis_index("s")
    wid = cid * 16 + sid

    def _ci(k):
        return k * _N_TILES + wid

    def _valid(k):
        return _ci(k) < _N_CHUNKS

    def _src_slice(k):
        return ei_hbm.at[0, pl.ds(_ci(k) * _CHUNK, _CHUNK)]

    def _dst_slice(k):
        return ei_hbm.at[1, pl.ds(_ci(k) * _CHUNK, _CHUNK)]

    def _slab_ok(j):
        return (j * 16 + sid) < _N_SLABS

    def _slab_r0(j):
        return pl.multiple_of((j * 16 + sid) * _WSLAB, 8)

    for k0 in range(_NBUF):
        pltpu.async_copy(_src_slice(k0), src_b[k0], isem[k0])

    def _zero_row(r, carry):
        for j in range(DIM // 16):
            rows_b[0][r, pl.ds(j * 16, 16)] = jnp.zeros((16,), jnp.float32)
        return carry

    lax.fori_loop(0, _WSLAB, _zero_row, 0)
    for j in range(_SLAB_ITERS):
        @pl.when(_slab_ok(j))
        def _():
            pltpu.async_copy(rows_b[0].at[pl.ds(0, _WSLAB)],
                             acc_sh.at[pl.ds(_slab_r0(j), _WSLAB)], gsem[0])
    for j in range(_SLAB_ITERS):
        @pl.when(_slab_ok(j))
        def _():
            pltpu.make_async_copy(rows_b[0].at[pl.ds(0, _WSLAB)],
                                  acc_sh.at[pl.ds(_slab_r0(j), _WSLAB)],
                                  gsem[0]).wait()
    plsc.subcore_barrier()

    def _edge_round(p, carry):
        for h in range(_NBUF):
            k = p * _NBUF + h
            b = h
            bp = (h + _NBUF - 1) % _NBUF

            @pl.when((k >= _NBUF) & _valid(k - _NBUF))
            def _():
                pltpu.make_async_copy(
                    table_hbm.at[pl.ds(0, _CHUNK)], rows_b[b], ssem[b]
                ).wait()

            @pl.when(_valid(k))
            def _():
                pltpu.make_async_copy(_src_slice(k), src_b[b], isem[b]).wait()
                pltpu.async_copy(table_hbm.at[src_b[b]], rows_b[b], gsem[b])
                pltpu.async_copy(_dst_slice(k), dst_b[b], ssem[b])

            @pl.when((k >= 1) & _valid(k - 1))
            def _():
                pltpu.make_async_copy(table_hbm.at[src_b[bp]], rows_b[bp],
                                      gsem[bp]).wait()

                @pl.when(_valid(k + _NBUF - 1))
                def _():
                    pltpu.async_copy(_src_slice(k + _NBUF - 1), src_b[bp],
                                     isem[bp])

                pltpu.make_async_copy(_dst_slice(k - 1), dst_b[bp],
                                      ssem[bp]).wait()
                pltpu.async_copy(rows_b[bp], acc_sh.at[dst_b[bp]], ssem[bp],
                                 add=True)

        return carry

    lax.fori_loop(0, (_ITERS + 2 * _NBUF) // _NBUF, _edge_round, 0)

    plsc.subcore_barrier()

    def _stage(j):
        return rows_b[j % _NBUF].at[pl.ds(0, _WSLAB)]

    def _fetch(j):
        return pltpu.make_async_copy(acc_sh.at[pl.ds(_slab_r0(j), _WSLAB)],
                                     _stage(j), gsem[j % _NBUF])

    def _store(j):
        return pltpu.make_async_copy(_stage(j),
                                     out_hbm.at[cid, pl.ds(_slab_r0(j), _WSLAB)],
                                     ssem[j % _NBUF])

    for j in range(_SLAB_ITERS + 1):
        if j < _SLAB_ITERS:
            if j >= _NBUF:
                @pl.when(_slab_ok(j - _NBUF))
                def _():
                    _store(j - _NBUF).wait()

            @pl.when(_slab_ok(j))
            def _():
                _fetch(j).start()

        if j >= 1:
            @pl.when(_slab_ok(j - 1))
            def _():
                _fetch(j - 1).wait()
                _store(j - 1).start()

    for j in range(_SLAB_ITERS - _NBUF, _SLAB_ITERS):
        @pl.when(_slab_ok(j))
        def _():
            _store(j).wait()


_BLK = 5000


def _mlp1_body(x_ref, a0_ref, a1_ref, w1_ref, b1_ref, w2_ref, b2_ref, o_ref):
    s = x_ref[...] + a0_ref[0] + a1_ref[0]
    h = jnp.dot(s, w1_ref[...], preferred_element_type=jnp.float32) + b1_ref[...]
    h = jnp.maximum(h, 0.0)
    o_ref[...] = jnp.dot(h, w2_ref[...], preferred_element_type=jnp.float32) + b2_ref[...]


def _mlp1(x, acc, W1, b1, W2, b2):
    grid = (N_NODES // _BLK,)
    return pl.pallas_call(
        _mlp1_body,
        grid=grid,
        in_specs=[
            pl.BlockSpec((_BLK, DIM), lambda i: (i, 0)),
            pl.BlockSpec((1, _BLK, DIM), lambda i: (0, i, 0)),
            pl.BlockSpec((1, _BLK, DIM), lambda i: (1, i, 0)),
            pl.BlockSpec((DIM, DIM), lambda i: (0, 0)),
            pl.BlockSpec((1, DIM), lambda i: (0, 0)),
            pl.BlockSpec((DIM, DIM), lambda i: (0, 0)),
            pl.BlockSpec((1, DIM), lambda i: (0, 0)),
        ],
        out_specs=pl.BlockSpec((_BLK, DIM), lambda i: (i, 0)),
        out_shape=jax.ShapeDtypeStruct((N_NODES, DIM), jnp.float32),
    )(x, acc, acc, W1, b1.reshape(1, DIM), W2, b2.reshape(1, DIM))


def _mlp2_body(h_ref, a0_ref, a1_ref, w3_ref, b3_ref, w4_ref, b4_ref,
               wf_ref, bf_ref, o_ref, acc_ref):
    s = h_ref[...] + a0_ref[0] + a1_ref[0]
    z = jnp.dot(s, w3_ref[...], preferred_element_type=jnp.float32) + b3_ref[...]
    part = jnp.sum(jnp.maximum(z, 0.0), axis=0, keepdims=True)

    @pl.when(pl.program_id(0) == 0)
    def _():
        acc_ref[...] = jnp.zeros_like(acc_ref)

    acc_ref[...] += part

    @pl.when(pl.program_id(0) == pl.num_programs(0) - 1)
    def _():
        pooled = (
            jnp.dot(acc_ref[...] * (1.0 / N_NODES), w4_ref[...],
                    preferred_element_type=jnp.float32)
            + b4_ref[...]
        )
        o_ref[...] = (
            jnp.dot(pooled, wf_ref[...], preferred_element_type=jnp.float32)
            + bf_ref[...]
        )


def _mlp2(h, acc, W3, b3, W4, b4, Wf, bf):
    grid = (N_NODES // _BLK,)
    return pl.pallas_call(
        _mlp2_body,
        grid=grid,
        in_specs=[
            pl.BlockSpec((_BLK, DIM), lambda i: (i, 0)),
            pl.BlockSpec((1, _BLK, DIM), lambda i: (0, i, 0)),
            pl.BlockSpec((1, _BLK, DIM), lambda i: (1, i, 0)),
            pl.BlockSpec((DIM, DIM), lambda i: (0, 0)),
            pl.BlockSpec((1, DIM), lambda i: (0, 0)),
            pl.BlockSpec((DIM, DIM), lambda i: (0, 0)),
            pl.BlockSpec((1, DIM), lambda i: (0, 0)),
            pl.BlockSpec((DIM, DIM), lambda i: (0, 0)),
            pl.BlockSpec((1, DIM), lambda i: (0, 0)),
        ],
        out_specs=pl.BlockSpec((1, DIM), lambda i: (0, 0)),
        out_shape=jax.ShapeDtypeStruct((1, DIM), jnp.float32),
        scratch_shapes=[pltpu.VMEM((1, DIM), jnp.float32)],
    )(h, acc, acc, W3, b3.reshape(1, DIM), W4, b4.reshape(1, DIM),
      Wf, bf.reshape(1, DIM))


def kernel(x, edge_index, W1, b1, W2, b2, W3, b3, W4, b4, Wf, bf):
    ei = edge_index.astype(jnp.int32)
    acc1 = _sc_agg(x, ei)
    h1 = _mlp1(x, acc1, W1, b1, W2, b2)
    acc2 = _sc_agg(h1, ei)
    out = _mlp2(h1, acc2, W3, b3, W4, b4, Wf, bf)
    return out.reshape(DIM)


# speedup vs baseline: 1.1858x; 1.0124x over previous
import functools

import jax
import jax.numpy as jnp
from jax import lax
from jax.experimental import pallas as pl
from jax.experimental.pallas import tpu as pltpu
from jax.experimental.pallas import tpu_sc as plsc

N_NODES = 10000
N_EDGES = 320000
DIM = 128

_N_TILES = 32
_CHUNK = 128
_N_CHUNKS = N_EDGES // _CHUNK
_NBUF = 3
_ITERS = 81
_WSLAB = 80
_N_SLABS = N_NODES // _WSLAB
_SLAB_ITERS = -(-_N_SLABS // 16)

_sc_mesh = plsc.VectorSubcoreMesh(core_axis_name="c", subcore_axis_name="s")


@functools.partial(
    pl.kernel,
    mesh=_sc_mesh,
    out_type=jax.ShapeDtypeStruct((2, N_NODES, DIM), jnp.float32),
    scratch_types=(
        [pltpu.VMEM((_CHUNK,), jnp.int32)] * _NBUF
        + [pltpu.VMEM((_CHUNK,), jnp.int32)] * _NBUF
        + [pltpu.VMEM((_CHUNK, DIM), jnp.float32)] * _NBUF
        + [pltpu.VMEM_SHARED((N_NODES, DIM), jnp.float32)]
        + [pltpu.SemaphoreType.DMA] * (3 * _NBUF + 1)
    ),
)
def _sc_agg(table_hbm, ei_hbm, out_hbm, *scr):
    src_b = scr[0:_NBUF]
    dst_b = scr[_NBUF:2 * _NBUF]
    rows_b = scr[2 * _NBUF:3 * _NBUF]
    acc_sh = scr[3 * _NBUF]
    isem = scr[3 * _NBUF + 1:3 * _NBUF + 1 + _NBUF]
    gsem = scr[3 * _NBUF + 1 + _NBUF:3 * _NBUF + 1 + 2 * _NBUF]
    ssem = scr[3 * _NBUF + 1 + 2 * _NBUF:3 * _NBUF + 1 + 3 * _NBUF]
    zsem = scr[3 * _NBUF + 1 + 3 * _NBUF]

    cid = lax.axis_index("c")
    sid = lax.axis_index("s")
    wid = cid * 16 + sid

    def _ci(k):
        return k * _N_TILES + wid

    def _valid(k):
        return _ci(k) < _N_CHUNKS

    def _src_slice(k):
        return ei_hbm.at[0, pl.ds(_ci(k) * _CHUNK, _CHUNK)]

    def _dst_slice(k):
        return ei_hbm.at[1, pl.ds(_ci(k) * _CHUNK, _CHUNK)]

    def _slab_ok(j):
        return (j * 16 + sid) < _N_SLABS

    def _slab_r0(j):
        return pl.multiple_of((j * 16 + sid) * _WSLAB, 8)

    for k0 in range(_NBUF):
        pltpu.async_copy(_src_slice(k0), src_b[k0], isem[k0])

    zbuf = rows_b[_NBUF - 1]

    def _zero_row(r, carry):
        for j in range(DIM // 16):
            zbuf[r, pl.ds(j * 16, 16)] = jnp.zeros((16,), jnp.float32)
        return carry

    lax.fori_loop(0, _WSLAB, _zero_row, 0)

    def _zfill(j):
        return pltpu.make_async_copy(zbuf.at[pl.ds(0, _WSLAB)],
                                     acc_sh.at[pl.ds(_slab_r0(j), _WSLAB)],
                                     zsem)

    for j in range(_SLAB_ITERS):
        @pl.when(_slab_ok(j))
        def _():
            _zfill(j).start()

    def _edge_round(p, carry):
        for h in range(_NBUF):
            k = p * _NBUF + h
            b = h
            bp = (h + _NBUF - 1) % _NBUF

            @pl.when((k >= _NBUF) & _valid(k - _NBUF))
            def _():
                pltpu.make_async_copy(
                    table_hbm.at[pl.ds(0, _CHUNK)], rows_b[b], ssem[b]
                ).wait()

            @pl.when(_valid(k))
            def _():
                pltpu.make_async_copy(_src_slice(k), src_b[b], isem[b]).wait()
                pltpu.async_copy(table_hbm.at[src_b[b]], rows_b[b], gsem[b])
                pltpu.async_copy(_dst_slice(k), dst_b[b], ssem[b])

            @pl.when((k >= 1) & _valid(k - 1))
            def _():
                pltpu.make_async_copy(table_hbm.at[src_b[bp]], rows_b[bp],
                                      gsem[bp]).wait()

                @pl.when(k == 1)
                def _():
                    for j in range(_SLAB_ITERS):
                        @pl.when(_slab_ok(j))
                        def _():
                            _zfill(j).wait()
                    plsc.subcore_barrier()

                @pl.when(_valid(k + _NBUF - 1))
                def _():
                    pltpu.async_copy(_src_slice(k + _NBUF - 1), src_b[bp],
                                     isem[bp])

                pltpu.make_async_copy(_dst_slice(k - 1), dst_b[bp],
                                      ssem[bp]).wait()
                pltpu.async_copy(rows_b[bp], acc_sh.at[dst_b[bp]], ssem[bp],
                                 add=True)

        return carry

    lax.fori_loop(0, (_ITERS + 2 * _NBUF) // _NBUF, _edge_round, 0)

    plsc.subcore_barrier()

    def _stage(j):
        return rows_b[j % _NBUF].at[pl.ds(0, _WSLAB)]

    def _fetch(j):
        return pltpu.make_async_copy(acc_sh.at[pl.ds(_slab_r0(j), _WSLAB)],
                                     _stage(j), gsem[j % _NBUF])

    def _store(j):
        return pltpu.make_async_copy(_stage(j),
                                     out_hbm.at[cid, pl.ds(_slab_r0(j), _WSLAB)],
                                     ssem[j % _NBUF])

    for j in range(_SLAB_ITERS + 1):
        if j < _SLAB_ITERS:
            if j >= _NBUF:
                @pl.when(_slab_ok(j - _NBUF))
                def _():
                    _store(j - _NBUF).wait()

            @pl.when(_slab_ok(j))
            def _():
                _fetch(j).start()

        if j >= 1:
            @pl.when(_slab_ok(j - 1))
            def _():
                _fetch(j - 1).wait()
                _store(j - 1).start()

    for j in range(_SLAB_ITERS - _NBUF, _SLAB_ITERS):
        @pl.when(_slab_ok(j))
        def _():
            _store(j).wait()


_BLK = 5000


def _mlp1_body(x_ref, a0_ref, a1_ref, w1_ref, b1_ref, w2_ref, b2_ref, o_ref):
    s = x_ref[...] + a0_ref[0] + a1_ref[0]
    h = jnp.dot(s, w1_ref[...], preferred_element_type=jnp.float32) + b1_ref[...]
    h = jnp.maximum(h, 0.0)
    o_ref[...] = jnp.dot(h, w2_ref[...], preferred_element_type=jnp.float32) + b2_ref[...]


def _mlp1(x, acc, W1, b1, W2, b2):
    grid = (N_NODES // _BLK,)
    return pl.pallas_call(
        _mlp1_body,
        grid=grid,
        in_specs=[
            pl.BlockSpec((_BLK, DIM), lambda i: (i, 0)),
            pl.BlockSpec((1, _BLK, DIM), lambda i: (0, i, 0)),
            pl.BlockSpec((1, _BLK, DIM), lambda i: (1, i, 0)),
            pl.BlockSpec((DIM, DIM), lambda i: (0, 0)),
            pl.BlockSpec((1, DIM), lambda i: (0, 0)),
            pl.BlockSpec((DIM, DIM), lambda i: (0, 0)),
            pl.BlockSpec((1, DIM), lambda i: (0, 0)),
        ],
        out_specs=pl.BlockSpec((_BLK, DIM), lambda i: (i, 0)),
        out_shape=jax.ShapeDtypeStruct((N_NODES, DIM), jnp.float32),
    )(x, acc, acc, W1, b1.reshape(1, DIM), W2, b2.reshape(1, DIM))


def _mlp2_body(h_ref, a0_ref, a1_ref, w3_ref, b3_ref, w4_ref, b4_ref,
               wf_ref, bf_ref, o_ref, acc_ref):
    s = h_ref[...] + a0_ref[0] + a1_ref[0]
    z = jnp.dot(s, w3_ref[...], preferred_element_type=jnp.float32) + b3_ref[...]
    part = jnp.sum(jnp.maximum(z, 0.0), axis=0, keepdims=True)

    @pl.when(pl.program_id(0) == 0)
    def _():
        acc_ref[...] = jnp.zeros_like(acc_ref)

    acc_ref[...] += part

    @pl.when(pl.program_id(0) == pl.num_programs(0) - 1)
    def _():
        pooled = (
            jnp.dot(acc_ref[...] * (1.0 / N_NODES), w4_ref[...],
                    preferred_element_type=jnp.float32)
            + b4_ref[...]
        )
        o_ref[...] = (
            jnp.dot(pooled, wf_ref[...], preferred_element_type=jnp.float32)
            + bf_ref[...]
        )


def _mlp2(h, acc, W3, b3, W4, b4, Wf, bf):
    grid = (N_NODES // _BLK,)
    return pl.pallas_call(
        _mlp2_body,
        grid=grid,
        in_specs=[
            pl.BlockSpec((_BLK, DIM), lambda i: (i, 0)),
            pl.BlockSpec((1, _BLK, DIM), lambda i: (0, i, 0)),
            pl.BlockSpec((1, _BLK, DIM), lambda i: (1, i, 0)),
            pl.BlockSpec((DIM, DIM), lambda i: (0, 0)),
            pl.BlockSpec((1, DIM), lambda i: (0, 0)),
            pl.BlockSpec((DIM, DIM), lambda i: (0, 0)),
            pl.BlockSpec((1, DIM), lambda i: (0, 0)),
            pl.BlockSpec((DIM, DIM), lambda i: (0, 0)),
            pl.BlockSpec((1, DIM), lambda i: (0, 0)),
        ],
        out_specs=pl.BlockSpec((1, DIM), lambda i: (0, 0)),
        out_shape=jax.ShapeDtypeStruct((1, DIM), jnp.float32),
        scratch_shapes=[pltpu.VMEM((1, DIM), jnp.float32)],
    )(h, acc, acc, W3, b3.reshape(1, DIM), W4, b4.reshape(1, DIM),
      Wf, bf.reshape(1, DIM))


def kernel(x, edge_index, W1, b1, W2, b2, W3, b3, W4, b4, Wf, bf):
    ei = edge_index.astype(jnp.int32)
    acc1 = _sc_agg(x, ei)
    h1 = _mlp1(x, acc1, W1, b1, W2, b2)
    acc2 = _sc_agg(h1, ei)
    out = _mlp2(h1, acc2, W3, b3, W4, b4, Wf, bf)
    return out.reshape(DIM)
